# baseline (speedup 1.0000x reference)
import functools

import jax
import jax.numpy as jnp
import numpy as np
from jax import lax
from jax.experimental import pallas as pl
from jax.experimental.pallas import tpu as pltpu
from jax.experimental.pallas import tpu_sc as plsc

N = 10000
E = 160000
D = 128
K = 16
EPS = 1e-15

NC = 2
NS = 16
NW = NC * NS
EPW = E // NW
C = 128
CHUNKS = (EPW + C - 1) // C
EPW_PAD = CHUNKS * C
NPAD = N + 8
RPT = 640
RSTRIDE = 624


def _sc_body(x_hbm, src_hbm, dst_hbm, ones_hbm, zag_hbm, zdg_hbm,
             aggp_hbm, degp_hbm,
             src_v, dst_v, rows_v, ones_v, agg_sh, deg_sh, gsem):
    cid = lax.axis_index("c")
    tid = lax.axis_index("s")
    r0 = tid * RSTRIDE

    pltpu.sync_copy(zag_hbm, agg_sh.at[pl.ds(r0, RPT)])
    pltpu.sync_copy(zdg_hbm, deg_sh.at[pl.ds(r0, RPT)])

    @pl.when(tid == NS - 1)
    def _():
        pltpu.sync_copy(zag_hbm.at[pl.ds(0, 8)], agg_sh.at[pl.ds(N, 8)])
        pltpu.sync_copy(zdg_hbm.at[pl.ds(0, 8)], deg_sh.at[pl.ds(N, 8)])

    pltpu.sync_copy(src_hbm.at[cid, tid], src_v)
    pltpu.sync_copy(dst_hbm.at[cid, tid], dst_v)
    pltpu.sync_copy(ones_hbm, ones_v)
    plsc.subcore_barrier()

    def chunk(j, carry):
        pltpu.async_copy(x_hbm.at[src_v.at[j]], rows_v, gsem).wait()
        pltpu.sync_copy(rows_v, agg_sh.at[dst_v.at[j]], add=True)
        pltpu.sync_copy(ones_v, deg_sh.at[dst_v.at[j]], add=True)
        return carry

    lax.fori_loop(0, CHUNKS, chunk, 0)
    plsc.subcore_barrier()

    pltpu.sync_copy(agg_sh.at[pl.ds(r0, RPT)], aggp_hbm.at[cid, pl.ds(r0, RPT)])
    pltpu.sync_copy(deg_sh.at[pl.ds(r0, RPT)], degp_hbm.at[cid, pl.ds(r0, RPT)])


_sc_agg = functools.partial(
    pl.kernel,
    out_type=(
        jax.ShapeDtypeStruct((NC, N, D), jnp.float32),
        jax.ShapeDtypeStruct((NC, N, K), jnp.float32),
    ),
    mesh=plsc.VectorSubcoreMesh(
        core_axis_name="c", subcore_axis_name="s",
        num_cores=NC, num_subcores=NS),
    compiler_params=pltpu.CompilerParams(use_tc_tiling_on_sc=False),
    scratch_types=[
        pltpu.VMEM((CHUNKS, C), jnp.int32),
        pltpu.VMEM((CHUNKS, C), jnp.int32),
        pltpu.VMEM((C, D), jnp.float32),
        pltpu.VMEM((C, K), jnp.float32),
        pltpu.VMEM_SHARED((NPAD, D), jnp.float32),
        pltpu.VMEM_SHARED((NPAD, K), jnp.float32),
        pltpu.SemaphoreType.DMA,
    ],
)(_sc_body)


R = 1000


def _tc_body(x_ref, aggp_ref, degp_ref, wl_ref, wr_ref, wm_ref,
             blb_ref, bmb_ref, out_ref, loss_ref, acc_ref):
    i = pl.program_id(0)
    agg = aggp_ref[0] + aggp_ref[1]
    deg = degp_ref[0, :, 0:1] + degp_ref[1, :, 0:1]
    mean = agg / jnp.maximum(deg, 1.0)
    h = mean @ wl_ref[...] + x_ref[...] @ wr_ref[...] + blb_ref[...]
    h = jnp.maximum(h, 0.0)
    s = h @ wm_ref[...] + bmb_ref[...]
    m = jnp.max(s, axis=-1, keepdims=True)
    e = jnp.exp(s - m)
    p = e / jnp.sum(e, axis=-1, keepdims=True)
    out_ref[...] = p

    @pl.when(i == 0)
    def _():
        acc_ref[...] = jnp.zeros_like(acc_ref)

    acc_ref[...] += jnp.sum(p * p, axis=0, keepdims=True)
    loss_ref[...] = (-jnp.sum(jnp.sqrt(acc_ref[...] + EPS), axis=1,
                              keepdims=True)
                     / np.float32(np.sqrt(N * K)))


_tc_dense = pl.pallas_call(
    _tc_body,
    grid=(N // R,),
    in_specs=[
        pl.BlockSpec((R, D), lambda i: (i, 0)),
        pl.BlockSpec((NC, R, D), lambda i: (0, i, 0)),
        pl.BlockSpec((NC, R, K), lambda i: (0, i, 0)),
        pl.BlockSpec((D, D), lambda i: (0, 0)),
        pl.BlockSpec((D, D), lambda i: (0, 0)),
        pl.BlockSpec((D, K), lambda i: (0, 0)),
        pl.BlockSpec((1, D), lambda i: (0, 0)),
        pl.BlockSpec((1, K), lambda i: (0, 0)),
    ],
    out_specs=[
        pl.BlockSpec((R, K), lambda i: (i, 0)),
        pl.BlockSpec((1, 1), lambda i: (0, 0)),
    ],
    out_shape=[
        jax.ShapeDtypeStruct((N, K), jnp.float32),
        jax.ShapeDtypeStruct((1, 1), jnp.float32),
    ],
    scratch_shapes=[pltpu.VMEM((1, K), jnp.float32)],
)


def kernel(x, edge_index, W_l, b_l, W_r, W_mlp, b_mlp):
    src = edge_index[0]
    dst = edge_index[1]
    srcp = jnp.concatenate(
        [src.reshape(NW, EPW), jnp.zeros((NW, EPW_PAD - EPW), jnp.int32)],
        axis=1).reshape(NC, NS, CHUNKS, C)
    dstp = jnp.concatenate(
        [dst.reshape(NW, EPW), jnp.full((NW, EPW_PAD - EPW), N, jnp.int32)],
        axis=1).reshape(NC, NS, CHUNKS, C)
    ones_c = jnp.ones((C, K), jnp.float32)
    zag = jnp.zeros((RPT, D), jnp.float32)
    zdg = jnp.zeros((RPT, K), jnp.float32)
    aggp, degp = _sc_agg(x, srcp, dstp, ones_c, zag, zdg)
    s_soft, loss = _tc_dense(x, aggp, degp, W_l.T, W_r.T, W_mlp.T,
                             b_l.reshape(1, D), b_mlp.reshape(1, K))
    return s_soft, loss[0, 0]


# speedup vs baseline: 1.4527x; 1.4527x over previous
import functools

import jax
import jax.numpy as jnp
import numpy as np
from jax import lax
from jax.experimental import pallas as pl
from jax.experimental.pallas import tpu as pltpu
from jax.experimental.pallas import tpu_sc as plsc

N = 10000
E = 160000
D = 128
HD = D // 2
K = 16
EPS = 1e-15

NC = 2
NS = 16
EPT = E // NS
C = 128
CHUNKS = (EPT + C - 1) // C
EPT_PAD = CHUNKS * C
NPAD = N + 8
RPT = 640
RSTRIDE = 624


def _sc_body(x2_hbm, src_hbm, dst_hbm, ones_hbm, zag_hbm, zdg_hbm,
             aggp_hbm, degp_hbm,
             src_v, dst_v, rows_v, ones_v, agg_sh, deg_sh, gsem0, gsem1):
    cid = lax.axis_index("c")
    tid = lax.axis_index("s")
    r0 = tid * RSTRIDE

    pltpu.sync_copy(zag_hbm, agg_sh.at[pl.ds(r0, RPT)])
    pltpu.sync_copy(zdg_hbm, deg_sh.at[pl.ds(r0, RPT)])

    @pl.when(tid == NS - 1)
    def _():
        pltpu.sync_copy(zag_hbm.at[pl.ds(0, 8)], agg_sh.at[pl.ds(N, 8)])
        pltpu.sync_copy(zdg_hbm.at[pl.ds(0, 8)], deg_sh.at[pl.ds(N, 8)])

    pltpu.sync_copy(src_hbm.at[cid, tid], src_v)
    pltpu.sync_copy(dst_hbm.at[tid], dst_v)
    pltpu.sync_copy(ones_hbm, ones_v)
    plsc.subcore_barrier()

    sems = (gsem0, gsem1)
    pend = [None, None]
    pend[0] = pltpu.async_copy(x2_hbm.at[src_v.at[0]], rows_v.at[0], sems[0])
    for q in range(CHUNKS):
        b = q % 2
        if q + 1 < CHUNKS:
            pend[1 - b] = pltpu.async_copy(
                x2_hbm.at[src_v.at[q + 1]], rows_v.at[1 - b], sems[1 - b])
        pend[b].wait()
        pltpu.sync_copy(rows_v.at[b], agg_sh.at[dst_v.at[q]], add=True)

        @pl.when(cid == q % 2)
        def _():
            pltpu.sync_copy(ones_v, deg_sh.at[dst_v.at[q]], add=True)

    plsc.subcore_barrier()

    pltpu.sync_copy(agg_sh.at[pl.ds(r0, RPT)], aggp_hbm.at[cid, pl.ds(r0, RPT)])
    pltpu.sync_copy(deg_sh.at[pl.ds(r0, RPT)], degp_hbm.at[cid, pl.ds(r0, RPT)])


_sc_agg = functools.partial(
    pl.kernel,
    out_type=(
        jax.ShapeDtypeStruct((NC, N, HD), jnp.float32),
        jax.ShapeDtypeStruct((NC, N, K), jnp.float32),
    ),
    mesh=plsc.VectorSubcoreMesh(
        core_axis_name="c", subcore_axis_name="s",
        num_cores=NC, num_subcores=NS),
    compiler_params=pltpu.CompilerParams(use_tc_tiling_on_sc=False),
    scratch_types=[
        pltpu.VMEM((CHUNKS, C), jnp.int32),
        pltpu.VMEM((CHUNKS, C), jnp.int32),
        pltpu.VMEM((2, C, HD), jnp.float32),
        pltpu.VMEM((C, K), jnp.float32),
        pltpu.VMEM_SHARED((NPAD, HD), jnp.float32),
        pltpu.VMEM_SHARED((NPAD, K), jnp.float32),
        pltpu.SemaphoreType.DMA,
        pltpu.SemaphoreType.DMA,
    ],
)(_sc_body)


R = 1000


def _tc_body(x_ref, aggp_ref, degp_ref, wl_ref, wr_ref, wm_ref,
             blb_ref, bmb_ref, out_ref, loss_ref, acc_ref):
    i = pl.program_id(0)
    agg = jnp.concatenate([aggp_ref[0], aggp_ref[1]], axis=-1)
    deg = degp_ref[0, :, 0:1] + degp_ref[1, :, 0:1]
    mean = agg / jnp.maximum(deg, 1.0)
    h = mean @ wl_ref[...] + x_ref[...] @ wr_ref[...] + blb_ref[...]
    h = jnp.maximum(h, 0.0)
    s = h @ wm_ref[...] + bmb_ref[...]
    m = jnp.max(s, axis=-1, keepdims=True)
    e = jnp.exp(s - m)
    p = e / jnp.sum(e, axis=-1, keepdims=True)
    out_ref[...] = p

    @pl.when(i == 0)
    def _():
        acc_ref[...] = jnp.zeros_like(acc_ref)

    acc_ref[...] += jnp.sum(p * p, axis=0, keepdims=True)
    loss_ref[...] = (-jnp.sum(jnp.sqrt(acc_ref[...] + EPS), axis=1,
                              keepdims=True)
                     / np.float32(np.sqrt(N * K)))


_tc_dense = pl.pallas_call(
    _tc_body,
    grid=(N // R,),
    in_specs=[
        pl.BlockSpec((R, D), lambda i: (i, 0)),
        pl.BlockSpec((NC, R, HD), lambda i: (0, i, 0)),
        pl.BlockSpec((NC, R, K), lambda i: (0, i, 0)),
        pl.BlockSpec((D, D), lambda i: (0, 0)),
        pl.BlockSpec((D, D), lambda i: (0, 0)),
        pl.BlockSpec((D, K), lambda i: (0, 0)),
        pl.BlockSpec((1, D), lambda i: (0, 0)),
        pl.BlockSpec((1, K), lambda i: (0, 0)),
    ],
    out_specs=[
        pl.BlockSpec((R, K), lambda i: (i, 0)),
        pl.BlockSpec((1, 1), lambda i: (0, 0)),
    ],
    out_shape=[
        jax.ShapeDtypeStruct((N, K), jnp.float32),
        jax.ShapeDtypeStruct((1, 1), jnp.float32),
    ],
    scratch_shapes=[pltpu.VMEM((1, K), jnp.float32)],
)


def kernel(x, edge_index, W_l, b_l, W_r, W_mlp, b_mlp):
    x2 = x.reshape(2 * N, HD)
    src = edge_index[0]
    dst = edge_index[1]
    pad = jnp.zeros((NS, EPT_PAD - EPT), jnp.int32)
    srcp = jnp.concatenate([src.reshape(NS, EPT), pad], axis=1)
    src2 = jnp.stack([2 * srcp, 2 * srcp + 1]).reshape(NC, NS, CHUNKS, C)
    dstp = jnp.concatenate(
        [dst.reshape(NS, EPT), jnp.full((NS, EPT_PAD - EPT), N, jnp.int32)],
        axis=1).reshape(NS, CHUNKS, C)
    ones_c = jnp.ones((C, K), jnp.float32)
    zag = jnp.zeros((RPT, HD), jnp.float32)
    zdg = jnp.zeros((RPT, K), jnp.float32)
    aggp, degp = _sc_agg(x2, src2, dstp, ones_c, zag, zdg)
    s_soft, loss = _tc_dense(x, aggp, degp, W_l.T, W_r.T, W_mlp.T,
                             b_l.reshape(1, D), b_mlp.reshape(1, K))
    return s_soft, loss[0, 0]


# speedup vs baseline: 1.4537x; 1.0007x over previous
import functools

import jax
import jax.numpy as jnp
import numpy as np
from jax import lax
from jax.experimental import pallas as pl
from jax.experimental.pallas import tpu as pltpu
from jax.experimental.pallas import tpu_sc as plsc

N = 10000
E = 160000
D = 128
HD = D // 2
K = 16
EPS = 1e-15

NC = 2
NS = 16
EPT = E // NS
C = 128
CHUNKS = (EPT + C - 1) // C
EPT_PAD = CHUNKS * C
NPAD = N + 8
RPT = 640
RSTRIDE = 624


def _sc_body(x2_hbm, src_hbm, dst_hbm, ones_hbm, zag_hbm, zdg_hbm,
             aggp_hbm, degp_hbm,
             src_v, dst_v, rows_v, ones_v, agg_sh, deg_sh, gsem0, gsem1,
             dsem):
    cid = lax.axis_index("c")
    tid = lax.axis_index("s")
    r0 = tid * RSTRIDE

    pltpu.sync_copy(zag_hbm, agg_sh.at[pl.ds(r0, RPT)])
    pltpu.sync_copy(zdg_hbm, deg_sh.at[pl.ds(r0, RPT)])

    @pl.when(tid == NS - 1)
    def _():
        pltpu.sync_copy(zag_hbm.at[pl.ds(0, 8)], agg_sh.at[pl.ds(N, 8)])
        pltpu.sync_copy(zdg_hbm.at[pl.ds(0, 8)], deg_sh.at[pl.ds(N, 8)])

    pltpu.sync_copy(src_hbm.at[cid, tid], src_v)
    pltpu.sync_copy(dst_hbm.at[tid], dst_v)
    pltpu.sync_copy(ones_hbm, ones_v)
    plsc.subcore_barrier()

    sems = (gsem0, gsem1)
    pend = [None, None]
    dlast = {0: None, 1: None}
    pend[0] = pltpu.async_copy(x2_hbm.at[src_v.at[0]], rows_v.at[0], sems[0])
    for q in range(CHUNKS):
        b = q % 2
        if q + 1 < CHUNKS:
            pend[1 - b] = pltpu.async_copy(
                x2_hbm.at[src_v.at[q + 1]], rows_v.at[1 - b], sems[1 - b])
        pend[b].wait()
        pltpu.sync_copy(rows_v.at[b], agg_sh.at[dst_v.at[q]], add=True)

        @pl.when(cid == b)
        def _(q=q, b=b):
            if dlast[b] is not None:
                dlast[b].wait()
            dlast[b] = pltpu.async_copy(
                ones_v, deg_sh.at[dst_v.at[q]], dsem, add=True)

    for p in (0, 1):
        if dlast[p] is not None:
            @pl.when(cid == p)
            def _(p=p):
                dlast[p].wait()

    plsc.subcore_barrier()

    pltpu.sync_copy(agg_sh.at[pl.ds(r0, RPT)], aggp_hbm.at[cid, pl.ds(r0, RPT)])
    pltpu.sync_copy(deg_sh.at[pl.ds(r0, RPT)], degp_hbm.at[cid, pl.ds(r0, RPT)])


_sc_agg = functools.partial(
    pl.kernel,
    out_type=(
        jax.ShapeDtypeStruct((NC, N, HD), jnp.float32),
        jax.ShapeDtypeStruct((NC, N, K), jnp.float32),
    ),
    mesh=plsc.VectorSubcoreMesh(
        core_axis_name="c", subcore_axis_name="s",
        num_cores=NC, num_subcores=NS),
    compiler_params=pltpu.CompilerParams(use_tc_tiling_on_sc=False),
    scratch_types=[
        pltpu.VMEM((CHUNKS, C), jnp.int32),
        pltpu.VMEM((CHUNKS, C), jnp.int32),
        pltpu.VMEM((2, C, HD), jnp.float32),
        pltpu.VMEM((C, K), jnp.float32),
        pltpu.VMEM_SHARED((NPAD, HD), jnp.float32),
        pltpu.VMEM_SHARED((NPAD, K), jnp.float32),
        pltpu.SemaphoreType.DMA,
        pltpu.SemaphoreType.DMA,
        pltpu.SemaphoreType.DMA,
    ],
)(_sc_body)


R = 1000


def _tc_body(x_ref, aggp_ref, degp_ref, wl_ref, wr_ref, wm_ref,
             blb_ref, bmb_ref, out_ref, loss_ref, acc_ref):
    i = pl.program_id(0)
    agg = jnp.concatenate([aggp_ref[0], aggp_ref[1]], axis=-1)
    deg = degp_ref[0, :, 0:1] + degp_ref[1, :, 0:1]
    mean = agg / jnp.maximum(deg, 1.0)
    h = mean @ wl_ref[...] + x_ref[...] @ wr_ref[...] + blb_ref[...]
    h = jnp.maximum(h, 0.0)
    s = h @ wm_ref[...] + bmb_ref[...]
    m = jnp.max(s, axis=-1, keepdims=True)
    e = jnp.exp(s - m)
    p = e / jnp.sum(e, axis=-1, keepdims=True)
    out_ref[...] = p

    @pl.when(i == 0)
    def _():
        acc_ref[...] = jnp.zeros_like(acc_ref)

    acc_ref[...] += jnp.sum(p * p, axis=0, keepdims=True)
    loss_ref[...] = (-jnp.sum(jnp.sqrt(acc_ref[...] + EPS), axis=1,
                              keepdims=True)
                     / np.float32(np.sqrt(N * K)))


_tc_dense = pl.pallas_call(
    _tc_body,
    grid=(N // R,),
    in_specs=[
        pl.BlockSpec((R, D), lambda i: (i, 0)),
        pl.BlockSpec((NC, R, HD), lambda i: (0, i, 0)),
        pl.BlockSpec((NC, R, K), lambda i: (0, i, 0)),
        pl.BlockSpec((D, D), lambda i: (0, 0)),
        pl.BlockSpec((D, D), lambda i: (0, 0)),
        pl.BlockSpec((D, K), lambda i: (0, 0)),
        pl.BlockSpec((1, D), lambda i: (0, 0)),
        pl.BlockSpec((1, K), lambda i: (0, 0)),
    ],
    out_specs=[
        pl.BlockSpec((R, K), lambda i: (i, 0)),
        pl.BlockSpec((1, 1), lambda i: (0, 0)),
    ],
    out_shape=[
        jax.ShapeDtypeStruct((N, K), jnp.float32),
        jax.ShapeDtypeStruct((1, 1), jnp.float32),
    ],
    scratch_shapes=[pltpu.VMEM((1, K), jnp.float32)],
)


def kernel(x, edge_index, W_l, b_l, W_r, W_mlp, b_mlp):
    x2 = x.reshape(2 * N, HD)
    src = edge_index[0]
    dst = edge_index[1]
    pad = jnp.zeros((NS, EPT_PAD - EPT), jnp.int32)
    srcp = jnp.concatenate([src.reshape(NS, EPT), pad], axis=1)
    src2 = jnp.stack([2 * srcp, 2 * srcp + 1]).reshape(NC, NS, CHUNKS, C)
    dstp = jnp.concatenate(
        [dst.reshape(NS, EPT), jnp.full((NS, EPT_PAD - EPT), N, jnp.int32)],
        axis=1).reshape(NS, CHUNKS, C)
    ones_c = jnp.ones((C, K), jnp.float32)
    zag = jnp.zeros((RPT, HD), jnp.float32)
    zdg = jnp.zeros((RPT, K), jnp.float32)
    aggp, degp = _sc_agg(x2, src2, dstp, ones_c, zag, zdg)
    s_soft, loss = _tc_dense(x, aggp, degp, W_l.T, W_r.T, W_mlp.T,
                             b_l.reshape(1, D), b_mlp.reshape(1, K))
    return s_soft, loss[0, 0]


# speedup vs baseline: 1.5645x; 1.0762x over previous
import functools

import jax
import jax.numpy as jnp
import numpy as np
from jax import lax
from jax.experimental import pallas as pl
from jax.experimental.pallas import tpu as pltpu
from jax.experimental.pallas import tpu_sc as plsc

N = 10000
E = 160000
D = 128
HD = D // 2
K = 16
EPS = 1e-15

NC = 2
NS = 16
EPT = E // NS
C = 128
CHUNKS = (EPT + C - 1) // C
EPT_PAD = CHUNKS * C
NPAD = N + 8
RPT = 640
RSTRIDE = 624


def _sc_body(x2_hbm, src_hbm, dst_hbm, ones_hbm, zag_hbm, zdg_hbm,
             aggp_hbm, degp_hbm,
             src_v, dst_v, rows_v, ones_v, agg_sh, deg_sh, gsem0, gsem1,
             dsem):
    cid = lax.axis_index("c")
    tid = lax.axis_index("s")
    r0 = tid * RSTRIDE

    pltpu.sync_copy(zag_hbm, agg_sh.at[pl.ds(r0, RPT)])
    pltpu.sync_copy(zdg_hbm, deg_sh.at[pl.ds(r0, RPT)])

    @pl.when(tid == NS - 1)
    def _():
        pltpu.sync_copy(zag_hbm.at[pl.ds(0, 8)], agg_sh.at[pl.ds(N, 8)])
        pltpu.sync_copy(zdg_hbm.at[pl.ds(0, 8)], deg_sh.at[pl.ds(N, 8)])

    pltpu.sync_copy(src_hbm.at[tid], src_v)
    pltpu.sync_copy(dst_hbm.at[tid], dst_v)
    pltpu.sync_copy(ones_hbm, ones_v)
    plsc.subcore_barrier()

    sems = (gsem0, gsem1)
    pend = [None, None]
    dlast = {0: None, 1: None}
    xc_hbm = x2_hbm.at[cid]
    pend[0] = pltpu.async_copy(xc_hbm.at[src_v.at[0]], rows_v.at[0], sems[0])
    for q in range(CHUNKS):
        b = q % 2
        if q + 1 < CHUNKS:
            pend[1 - b] = pltpu.async_copy(
                xc_hbm.at[src_v.at[q + 1]], rows_v.at[1 - b], sems[1 - b])
        pend[b].wait()
        pltpu.sync_copy(rows_v.at[b], agg_sh.at[dst_v.at[q]], add=True)

        @pl.when(cid == b)
        def _(q=q, b=b):
            if dlast[b] is not None:
                dlast[b].wait()
            dlast[b] = pltpu.async_copy(
                ones_v, deg_sh.at[dst_v.at[q]], dsem, add=True)

    for p in (0, 1):
        if dlast[p] is not None:
            @pl.when(cid == p)
            def _(p=p):
                dlast[p].wait()

    plsc.subcore_barrier()

    pltpu.sync_copy(agg_sh.at[pl.ds(r0, RPT)], aggp_hbm.at[cid, pl.ds(r0, RPT)])
    pltpu.sync_copy(deg_sh.at[pl.ds(r0, RPT)], degp_hbm.at[cid, pl.ds(r0, RPT)])


_sc_agg = functools.partial(
    pl.kernel,
    out_type=(
        jax.ShapeDtypeStruct((NC, N, HD), jnp.float32),
        jax.ShapeDtypeStruct((NC, N, K), jnp.float32),
    ),
    mesh=plsc.VectorSubcoreMesh(
        core_axis_name="c", subcore_axis_name="s",
        num_cores=NC, num_subcores=NS),
    compiler_params=pltpu.CompilerParams(use_tc_tiling_on_sc=False),
    scratch_types=[
        pltpu.VMEM((CHUNKS, C), jnp.int32),
        pltpu.VMEM((CHUNKS, C), jnp.int32),
        pltpu.VMEM((2, C, HD), jnp.float32),
        pltpu.VMEM((C, K), jnp.float32),
        pltpu.VMEM_SHARED((NPAD, HD), jnp.float32),
        pltpu.VMEM_SHARED((NPAD, K), jnp.float32),
        pltpu.SemaphoreType.DMA,
        pltpu.SemaphoreType.DMA,
        pltpu.SemaphoreType.DMA,
    ],
)(_sc_body)


R = 1000


def _tc_body(x_ref, aggp_ref, degp_ref, wl_ref, wr_ref, wm_ref,
             blb_ref, bmb_ref, out_ref, loss_ref, acc_ref):
    i = pl.program_id(0)
    agg = jnp.concatenate([aggp_ref[0], aggp_ref[1]], axis=-1)
    deg = degp_ref[0, :, 0:1] + degp_ref[1, :, 0:1]
    mean = agg / jnp.maximum(deg, 1.0)
    h = mean @ wl_ref[...] + x_ref[...] @ wr_ref[...] + blb_ref[...]
    h = jnp.maximum(h, 0.0)
    s = h @ wm_ref[...] + bmb_ref[...]
    m = jnp.max(s, axis=-1, keepdims=True)
    e = jnp.exp(s - m)
    p = e / jnp.sum(e, axis=-1, keepdims=True)
    out_ref[...] = p

    @pl.when(i == 0)
    def _():
        acc_ref[...] = jnp.zeros_like(acc_ref)

    acc_ref[...] += jnp.sum(p * p, axis=0, keepdims=True)
    loss_ref[...] = (-jnp.sum(jnp.sqrt(acc_ref[...] + EPS), axis=1,
                              keepdims=True)
                     / np.float32(np.sqrt(N * K)))


_tc_dense = pl.pallas_call(
    _tc_body,
    grid=(N // R,),
    in_specs=[
        pl.BlockSpec((R, D), lambda i: (i, 0)),
        pl.BlockSpec((NC, R, HD), lambda i: (0, i, 0)),
        pl.BlockSpec((NC, R, K), lambda i: (0, i, 0)),
        pl.BlockSpec((D, D), lambda i: (0, 0)),
        pl.BlockSpec((D, D), lambda i: (0, 0)),
        pl.BlockSpec((D, K), lambda i: (0, 0)),
        pl.BlockSpec((1, D), lambda i: (0, 0)),
        pl.BlockSpec((1, K), lambda i: (0, 0)),
    ],
    out_specs=[
        pl.BlockSpec((R, K), lambda i: (i, 0)),
        pl.BlockSpec((1, 1), lambda i: (0, 0)),
    ],
    out_shape=[
        jax.ShapeDtypeStruct((N, K), jnp.float32),
        jax.ShapeDtypeStruct((1, 1), jnp.float32),
    ],
    scratch_shapes=[pltpu.VMEM((1, K), jnp.float32)],
)


def kernel(x, edge_index, W_l, b_l, W_r, W_mlp, b_mlp):
    x_cols = jnp.transpose(x.reshape(N, NC, HD), (1, 0, 2))
    src = edge_index[0]
    dst = edge_index[1]
    pad = jnp.zeros((NS, EPT_PAD - EPT), jnp.int32)
    srcp = jnp.concatenate([src.reshape(NS, EPT), pad],
                           axis=1).reshape(NS, CHUNKS, C)
    dstp = jnp.concatenate(
        [dst.reshape(NS, EPT), jnp.full((NS, EPT_PAD - EPT), N, jnp.int32)],
        axis=1).reshape(NS, CHUNKS, C)
    ones_c = jnp.ones((C, K), jnp.float32)
    zag = jnp.zeros((RPT, HD), jnp.float32)
    zdg = jnp.zeros((RPT, K), jnp.float32)
    aggp, degp = _sc_agg(x_cols, srcp, dstp, ones_c, zag, zdg)
    s_soft, loss = _tc_dense(x, aggp, degp, W_l.T, W_r.T, W_mlp.T,
                             b_l.reshape(1, D), b_mlp.reshape(1, K))
    return s_soft, loss[0, 0]


# speedup vs baseline: 1.6617x; 1.0621x over previous
import functools

import jax
import jax.numpy as jnp
import numpy as np
from jax import lax
from jax.experimental import pallas as pl
from jax.experimental.pallas import tpu as pltpu
from jax.experimental.pallas import tpu_sc as plsc

N = 10000
E = 160000
D = 128
HD = D // 2
K = 16
EPS = 1e-15

NC = 2
NS = 16
EPT = E // NS
C = 128
CHUNKS = (EPT + C - 1) // C
EPT_PAD = CHUNKS * C
NBUF = 4
NPAD = N + 8
RPT = 640
RSTRIDE = 624


def _sc_body(x2_hbm, src_hbm, dst_hbm, ones_hbm, zag_hbm, zdg_hbm,
             aggp_hbm, degp_hbm,
             src_v, dst_v, rows_v, ones_v, agg_sh, deg_sh, gsem0, gsem1,
             gsem2, gsem3, dsem):
    cid = lax.axis_index("c")
    tid = lax.axis_index("s")
    r0 = tid * RSTRIDE

    pltpu.sync_copy(zag_hbm, agg_sh.at[pl.ds(r0, RPT)])
    pltpu.sync_copy(zdg_hbm, deg_sh.at[pl.ds(r0, RPT)])

    @pl.when(tid == NS - 1)
    def _():
        pltpu.sync_copy(zag_hbm.at[pl.ds(0, 8)], agg_sh.at[pl.ds(N, 8)])
        pltpu.sync_copy(zdg_hbm.at[pl.ds(0, 8)], deg_sh.at[pl.ds(N, 8)])

    pltpu.sync_copy(src_hbm.at[tid], src_v)
    pltpu.sync_copy(dst_hbm.at[tid], dst_v)
    pltpu.sync_copy(ones_hbm, ones_v)
    plsc.subcore_barrier()

    sems = (gsem0, gsem1, gsem2, gsem3)
    pend = [None] * NBUF
    dlast = {0: None, 1: None}
    xc_hbm = x2_hbm.at[cid]
    for j in range(NBUF - 1):
        pend[j] = pltpu.async_copy(
            xc_hbm.at[src_v.at[j]], rows_v.at[j], sems[j])
    for q in range(CHUNKS):
        b = q % NBUF
        if q + NBUF - 1 < CHUNKS:
            bn = (q + NBUF - 1) % NBUF
            pend[bn] = pltpu.async_copy(
                xc_hbm.at[src_v.at[q + NBUF - 1]], rows_v.at[bn], sems[bn])
        pend[b].wait()
        pltpu.sync_copy(rows_v.at[b], agg_sh.at[dst_v.at[q]], add=True)

        p = q % 2
        @pl.when(cid == p)
        def _(q=q, p=p):
            if dlast[p] is not None:
                dlast[p].wait()
            dlast[p] = pltpu.async_copy(
                ones_v, deg_sh.at[dst_v.at[q]], dsem, add=True)

    for p in (0, 1):
        if dlast[p] is not None:
            @pl.when(cid == p)
            def _(p=p):
                dlast[p].wait()

    plsc.subcore_barrier()

    pltpu.sync_copy(agg_sh.at[pl.ds(r0, RPT)], aggp_hbm.at[cid, pl.ds(r0, RPT)])
    pltpu.sync_copy(deg_sh.at[pl.ds(r0, RPT)], degp_hbm.at[cid, pl.ds(r0, RPT)])


_sc_agg = functools.partial(
    pl.kernel,
    out_type=(
        jax.ShapeDtypeStruct((NC, N, HD), jnp.float32),
        jax.ShapeDtypeStruct((NC, N, K), jnp.float32),
    ),
    mesh=plsc.VectorSubcoreMesh(
        core_axis_name="c", subcore_axis_name="s",
        num_cores=NC, num_subcores=NS),
    compiler_params=pltpu.CompilerParams(use_tc_tiling_on_sc=False),
    scratch_types=[
        pltpu.VMEM((CHUNKS, C), jnp.int32),
        pltpu.VMEM((CHUNKS, C), jnp.int32),
        pltpu.VMEM((NBUF, C, HD), jnp.float32),
        pltpu.VMEM((C, K), jnp.float32),
        pltpu.VMEM_SHARED((NPAD, HD), jnp.float32),
        pltpu.VMEM_SHARED((NPAD, K), jnp.float32),
        pltpu.SemaphoreType.DMA,
        pltpu.SemaphoreType.DMA,
        pltpu.SemaphoreType.DMA,
        pltpu.SemaphoreType.DMA,
        pltpu.SemaphoreType.DMA,
    ],
)(_sc_body)


R = 1000


def _tc_body(x_ref, aggp_ref, degp_ref, wl_ref, wr_ref, wm_ref,
             blb_ref, bmb_ref, out_ref, loss_ref, acc_ref):
    i = pl.program_id(0)
    agg = jnp.concatenate([aggp_ref[0], aggp_ref[1]], axis=-1)
    deg = degp_ref[0, :, 0:1] + degp_ref[1, :, 0:1]
    mean = agg / jnp.maximum(deg, 1.0)
    h = mean @ wl_ref[...] + x_ref[...] @ wr_ref[...] + blb_ref[...]
    h = jnp.maximum(h, 0.0)
    s = h @ wm_ref[...] + bmb_ref[...]
    m = jnp.max(s, axis=-1, keepdims=True)
    e = jnp.exp(s - m)
    p = e / jnp.sum(e, axis=-1, keepdims=True)
    out_ref[...] = p

    @pl.when(i == 0)
    def _():
        acc_ref[...] = jnp.zeros_like(acc_ref)

    acc_ref[...] += jnp.sum(p * p, axis=0, keepdims=True)
    loss_ref[...] = (-jnp.sum(jnp.sqrt(acc_ref[...] + EPS), axis=1,
                              keepdims=True)
                     / np.float32(np.sqrt(N * K)))


_tc_dense = pl.pallas_call(
    _tc_body,
    grid=(N // R,),
    in_specs=[
        pl.BlockSpec((R, D), lambda i: (i, 0)),
        pl.BlockSpec((NC, R, HD), lambda i: (0, i, 0)),
        pl.BlockSpec((NC, R, K), lambda i: (0, i, 0)),
        pl.BlockSpec((D, D), lambda i: (0, 0)),
        pl.BlockSpec((D, D), lambda i: (0, 0)),
        pl.BlockSpec((D, K), lambda i: (0, 0)),
        pl.BlockSpec((1, D), lambda i: (0, 0)),
        pl.BlockSpec((1, K), lambda i: (0, 0)),
    ],
    out_specs=[
        pl.BlockSpec((R, K), lambda i: (i, 0)),
        pl.BlockSpec((1, 1), lambda i: (0, 0)),
    ],
    out_shape=[
        jax.ShapeDtypeStruct((N, K), jnp.float32),
        jax.ShapeDtypeStruct((1, 1), jnp.float32),
    ],
    scratch_shapes=[pltpu.VMEM((1, K), jnp.float32)],
)


def kernel(x, edge_index, W_l, b_l, W_r, W_mlp, b_mlp):
    x_cols = jnp.transpose(x.reshape(N, NC, HD), (1, 0, 2))
    src = edge_index[0]
    dst = edge_index[1]
    pad = jnp.zeros((NS, EPT_PAD - EPT), jnp.int32)
    srcp = jnp.concatenate([src.reshape(NS, EPT), pad],
                           axis=1).reshape(NS, CHUNKS, C)
    dstp = jnp.concatenate(
        [dst.reshape(NS, EPT), jnp.full((NS, EPT_PAD - EPT), N, jnp.int32)],
        axis=1).reshape(NS, CHUNKS, C)
    ones_c = jnp.ones((C, K), jnp.float32)
    zag = jnp.zeros((RPT, HD), jnp.float32)
    zdg = jnp.zeros((RPT, K), jnp.float32)
    aggp, degp = _sc_agg(x_cols, srcp, dstp, ones_c, zag, zdg)
    s_soft, loss = _tc_dense(x, aggp, degp, W_l.T, W_r.T, W_mlp.T,
                             b_l.reshape(1, D), b_mlp.reshape(1, K))
    return s_soft, loss[0, 0]


# speedup vs baseline: 1.6691x; 1.0045x over previous
import functools

import jax
import jax.numpy as jnp
import numpy as np
from jax import lax
from jax.experimental import pallas as pl
from jax.experimental.pallas import tpu as pltpu
from jax.experimental.pallas import tpu_sc as plsc

N = 10000
E = 160000
D = 128
HD = D // 2
K = 16
EPS = 1e-15

NC = 2
NS = 16
EPT = E // NS
C = 128
CHUNKS = (EPT + C - 1) // C
EPT_PAD = CHUNKS * C
NBUF = 6
NPAD = N + 8
RPT = 640
RSTRIDE = 624


def _sc_body(x2_hbm, src_hbm, dst_hbm, ones_hbm, zag_hbm, zdg_hbm,
             aggp_hbm, degp_hbm,
             src_v, dst_v, rows_v, ones_v, agg_sh, deg_sh, gsem0, gsem1,
             gsem2, gsem3, gsem4, gsem5, dsem):
    cid = lax.axis_index("c")
    tid = lax.axis_index("s")
    r0 = tid * RSTRIDE

    pltpu.sync_copy(zag_hbm, agg_sh.at[pl.ds(r0, RPT)])
    pltpu.sync_copy(zdg_hbm, deg_sh.at[pl.ds(r0, RPT)])

    @pl.when(tid == NS - 1)
    def _():
        pltpu.sync_copy(zag_hbm.at[pl.ds(0, 8)], agg_sh.at[pl.ds(N, 8)])
        pltpu.sync_copy(zdg_hbm.at[pl.ds(0, 8)], deg_sh.at[pl.ds(N, 8)])

    pltpu.sync_copy(src_hbm.at[tid], src_v)
    pltpu.sync_copy(dst_hbm.at[tid], dst_v)
    pltpu.sync_copy(ones_hbm, ones_v)
    plsc.subcore_barrier()

    sems = (gsem0, gsem1, gsem2, gsem3, gsem4, gsem5)
    pend = [None] * NBUF
    dlast = {0: None, 1: None}
    xc_hbm = x2_hbm.at[cid]
    for j in range(NBUF - 1):
        pend[j] = pltpu.async_copy(
            xc_hbm.at[src_v.at[j]], rows_v.at[j], sems[j])
    for q in range(CHUNKS):
        b = q % NBUF
        if q + NBUF - 1 < CHUNKS:
            bn = (q + NBUF - 1) % NBUF
            pend[bn] = pltpu.async_copy(
                xc_hbm.at[src_v.at[q + NBUF - 1]], rows_v.at[bn], sems[bn])
        pend[b].wait()
        pltpu.sync_copy(rows_v.at[b], agg_sh.at[dst_v.at[q]], add=True)

        p = q % 2
        @pl.when(cid == p)
        def _(q=q, p=p):
            if dlast[p] is not None:
                dlast[p].wait()
            dlast[p] = pltpu.async_copy(
                ones_v, deg_sh.at[dst_v.at[q]], dsem, add=True)

    for p in (0, 1):
        if dlast[p] is not None:
            @pl.when(cid == p)
            def _(p=p):
                dlast[p].wait()

    plsc.subcore_barrier()

    pltpu.sync_copy(agg_sh.at[pl.ds(r0, RPT)], aggp_hbm.at[cid, pl.ds(r0, RPT)])
    pltpu.sync_copy(deg_sh.at[pl.ds(r0, RPT)], degp_hbm.at[cid, pl.ds(r0, RPT)])


_sc_agg = functools.partial(
    pl.kernel,
    out_type=(
        jax.ShapeDtypeStruct((NC, N, HD), jnp.float32),
        jax.ShapeDtypeStruct((NC, N, K), jnp.float32),
    ),
    mesh=plsc.VectorSubcoreMesh(
        core_axis_name="c", subcore_axis_name="s",
        num_cores=NC, num_subcores=NS),
    compiler_params=pltpu.CompilerParams(use_tc_tiling_on_sc=False),
    scratch_types=[
        pltpu.VMEM((CHUNKS, C), jnp.int32),
        pltpu.VMEM((CHUNKS, C), jnp.int32),
        pltpu.VMEM((NBUF, C, HD), jnp.float32),
        pltpu.VMEM((C, K), jnp.float32),
        pltpu.VMEM_SHARED((NPAD, HD), jnp.float32),
        pltpu.VMEM_SHARED((NPAD, K), jnp.float32),
        pltpu.SemaphoreType.DMA,
        pltpu.SemaphoreType.DMA,
        pltpu.SemaphoreType.DMA,
        pltpu.SemaphoreType.DMA,
        pltpu.SemaphoreType.DMA,
        pltpu.SemaphoreType.DMA,
        pltpu.SemaphoreType.DMA,
    ],
)(_sc_body)


R = 1000


def _tc_body(x_ref, aggp_ref, degp_ref, wl_ref, wr_ref, wm_ref,
             blb_ref, bmb_ref, out_ref, loss_ref, acc_ref):
    i = pl.program_id(0)
    agg = jnp.concatenate([aggp_ref[0], aggp_ref[1]], axis=-1)
    deg = degp_ref[0, :, 0:1] + degp_ref[1, :, 0:1]
    mean = agg / jnp.maximum(deg, 1.0)
    h = mean @ wl_ref[...] + x_ref[...] @ wr_ref[...] + blb_ref[...]
    h = jnp.maximum(h, 0.0)
    s = h @ wm_ref[...] + bmb_ref[...]
    m = jnp.max(s, axis=-1, keepdims=True)
    e = jnp.exp(s - m)
    p = e / jnp.sum(e, axis=-1, keepdims=True)
    out_ref[...] = p

    @pl.when(i == 0)
    def _():
        acc_ref[...] = jnp.zeros_like(acc_ref)

    acc_ref[...] += jnp.sum(p * p, axis=0, keepdims=True)
    loss_ref[...] = (-jnp.sum(jnp.sqrt(acc_ref[...] + EPS), axis=1,
                              keepdims=True)
                     / np.float32(np.sqrt(N * K)))


_tc_dense = pl.pallas_call(
    _tc_body,
    grid=(N // R,),
    in_specs=[
        pl.BlockSpec((R, D), lambda i: (i, 0)),
        pl.BlockSpec((NC, R, HD), lambda i: (0, i, 0)),
        pl.BlockSpec((NC, R, K), lambda i: (0, i, 0)),
        pl.BlockSpec((D, D), lambda i: (0, 0)),
        pl.BlockSpec((D, D), lambda i: (0, 0)),
        pl.BlockSpec((D, K), lambda i: (0, 0)),
        pl.BlockSpec((1, D), lambda i: (0, 0)),
        pl.BlockSpec((1, K), lambda i: (0, 0)),
    ],
    out_specs=[
        pl.BlockSpec((R, K), lambda i: (i, 0)),
        pl.BlockSpec((1, 1), lambda i: (0, 0)),
    ],
    out_shape=[
        jax.ShapeDtypeStruct((N, K), jnp.float32),
        jax.ShapeDtypeStruct((1, 1), jnp.float32),
    ],
    scratch_shapes=[pltpu.VMEM((1, K), jnp.float32)],
)


def kernel(x, edge_index, W_l, b_l, W_r, W_mlp, b_mlp):
    x_cols = jnp.transpose(x.reshape(N, NC, HD), (1, 0, 2))
    src = edge_index[0]
    dst = edge_index[1]
    pad = jnp.zeros((NS, EPT_PAD - EPT), jnp.int32)
    srcp = jnp.concatenate([src.reshape(NS, EPT), pad],
                           axis=1).reshape(NS, CHUNKS, C)
    dstp = jnp.concatenate(
        [dst.reshape(NS, EPT), jnp.full((NS, EPT_PAD - EPT), N, jnp.int32)],
        axis=1).reshape(NS, CHUNKS, C)
    ones_c = jnp.ones((C, K), jnp.float32)
    zag = jnp.zeros((RPT, HD), jnp.float32)
    zdg = jnp.zeros((RPT, K), jnp.float32)
    aggp, degp = _sc_agg(x_cols, srcp, dstp, ones_c, zag, zdg)
    s_soft, loss = _tc_dense(x, aggp, degp, W_l.T, W_r.T, W_mlp.T,
                             b_l.reshape(1, D), b_mlp.reshape(1, K))
    return s_soft, loss[0, 0]
